# baseline (speedup 1.0000x reference)
import functools

import jax
import jax.numpy as jnp
from jax import lax
from jax.experimental import pallas as pl
from jax.experimental.pallas import tpu as pltpu
from jax.experimental.pallas import tpu_sc as plsc

N = 10000
E = 160000
IN_DIM = 256
HID = 512
LAYERS = 3
NUM_CLASSES = 3

CW = 128
NCHUNK = HID // CW
EB = 128
NTILES = 16
TPT = E // NTILES
NJ = 78
TAIL = TPT - NJ * EB
NJD = 39
TAILD = E // 32 - NJD * EB
STRIPE = 632
LAST_STRIPE = N - 15 * STRIPE
ZROWS = 320
MB = 2000


def _agg_body(*refs, with_deg):
    if with_deg:
        (hT, src3, dst3, srct, dstt, z128, dstd, dstdt, ones128, agg, degs,
         src_v, dst_v, srct_v, dstt_v, gbuf, slab, sem,
         dstd_v, dstdt_v) = refs
    else:
        (hT, src3, dst3, srct, dstt, z128, agg,
         src_v, dst_v, srct_v, dstt_v, gbuf, slab, sem) = refs
    c = lax.axis_index("c")
    s = lax.axis_index("s")
    pltpu.sync_copy(src3.at[s], src_v)
    pltpu.sync_copy(dst3.at[s], dst_v)
    pltpu.sync_copy(srct.at[s], srct_v)
    pltpu.sync_copy(dstt.at[s], dstt_v)

    base = s * STRIPE

    def zero_stripes(slab):
        def zero(sz):
            def _z():
                pltpu.sync_copy(z128, slab.at[pl.ds(base, ZROWS)])
                pltpu.sync_copy(z128.at[pl.ds(0, sz - ZROWS)],
                                slab.at[pl.ds(base + ZROWS, sz - ZROWS)])
            return _z
        pl.when(s < 15)(zero(STRIPE))
        pl.when(s == 15)(zero(LAST_STRIPE))

    def copy_stripes(slab, out):
        def copyout(sz):
            return lambda: pltpu.sync_copy(slab.at[pl.ds(base, sz)],
                                           out.at[pl.ds(base, sz)])
        pl.when(s < 15)(copyout(STRIPE))
        pl.when(s == 15)(copyout(LAST_STRIPE))

    for cc in range(NCHUNK):
        @pl.when(cc % 2 == c)
        def _process(cc=cc):
            zero_stripes(slab)
            plsc.subcore_barrier()

            def step(j, carry):
                pltpu.async_copy(hT.at[cc].at[src_v.at[j]], gbuf,
                                 sem).wait()
                pltpu.sync_copy(gbuf, slab.at[dst_v.at[j]], add=True)
                return carry

            lax.fori_loop(0, NJ, step, 0)
            pltpu.async_copy(hT.at[cc].at[srct_v.at[0]],
                             gbuf.at[pl.ds(0, TAIL)], sem).wait()
            pltpu.sync_copy(gbuf.at[pl.ds(0, TAIL)],
                            slab.at[dstt_v.at[0]], add=True)
            plsc.subcore_barrier()

            copy_stripes(slab, agg.at[cc])
            plsc.subcore_barrier()

    if with_deg:
        wid = c * NTILES + s
        pltpu.sync_copy(dstd.at[wid], dstd_v)
        pltpu.sync_copy(dstdt.at[wid], dstdt_v)
        pltpu.sync_copy(ones128, gbuf)
        zero_stripes(slab)
        plsc.subcore_barrier()

        def dstep(j, carry):
            pltpu.sync_copy(gbuf, slab.at[dstd_v.at[j]], add=True)
            return carry

        lax.fori_loop(0, NJD, dstep, 0)
        pltpu.sync_copy(gbuf.at[pl.ds(0, TAILD)],
                        slab.at[dstdt_v.at[0]], add=True)
        plsc.subcore_barrier()
        copy_stripes(slab, degs.at[c])


def _make_agg(with_deg):
    mesh = plsc.VectorSubcoreMesh(core_axis_name="c", subcore_axis_name="s")
    out_type = [jax.ShapeDtypeStruct((NCHUNK, N, CW), jnp.float32)]
    scratch = [
        pltpu.VMEM((NJ, EB), jnp.int32),
        pltpu.VMEM((NJ, EB), jnp.int32),
        pltpu.VMEM((1, TAIL), jnp.int32),
        pltpu.VMEM((1, TAIL), jnp.int32),
        pltpu.VMEM((EB, CW), jnp.float32),
        pltpu.VMEM_SHARED((N, CW), jnp.float32),
        pltpu.SemaphoreType.DMA,
    ]
    if with_deg:
        out_type.append(jax.ShapeDtypeStruct((2, N, CW), jnp.float32))
        scratch.append(pltpu.VMEM((NJD, EB), jnp.int32))
        scratch.append(pltpu.VMEM((1, TAILD), jnp.int32))
    return pl.kernel(
        functools.partial(_agg_body, with_deg=with_deg),
        mesh=mesh,
        out_type=out_type if with_deg else out_type[0],
        scratch_types=scratch,
    )


def _in_proj_body(x_ref, w_ref, b_ref, o_ref):
    h = jnp.dot(x_ref[...], w_ref[...],
                preferred_element_type=jnp.float32,
                precision=lax.Precision.DEFAULT) + b_ref[...]
    for cc in range(NCHUNK):
        o_ref[cc] = h[:, cc * CW:(cc + 1) * CW]


def _in_proj(x, w, b2):
    return pl.pallas_call(
        _in_proj_body,
        grid=(N // MB,),
        in_specs=[
            pl.BlockSpec((MB, IN_DIM), lambda r: (r, 0)),
            pl.BlockSpec((IN_DIM, HID), lambda r: (0, 0)),
            pl.BlockSpec((1, HID), lambda r: (0, 0)),
        ],
        out_specs=pl.BlockSpec((NCHUNK, MB, CW), lambda r: (0, r, 0)),
        out_shape=jax.ShapeDtypeStruct((NCHUNK, N, CW), jnp.float32),
    )(x, w, b2)


def _layer_math(h_ref, agg_ref, deg_ref, ws_ref, wn_ref, bs_ref, bn_ref,
                g_ref, be_ref):
    deg = jnp.maximum(deg_ref[0][:, 0:1] + deg_ref[1][:, 0:1], 1.0)
    recip = 1.0 / deg
    wsv = ws_ref[0]
    wnv = wn_ref[0]
    z = bs_ref[0] + bn_ref[0]
    for cc in range(NCHUNK):
        z = z + jnp.dot(h_ref[cc], wsv[cc * CW:(cc + 1) * CW, :],
                        preferred_element_type=jnp.float32,
                        precision=lax.Precision.DEFAULT)
        z = z + jnp.dot(agg_ref[cc] * recip, wnv[cc * CW:(cc + 1) * CW, :],
                        preferred_element_type=jnp.float32,
                        precision=lax.Precision.DEFAULT)
    mu = jnp.mean(z, axis=1, keepdims=True)
    d = z - mu
    var = jnp.mean(d * d, axis=1, keepdims=True)
    zn = d * lax.rsqrt(var + 1e-5) * g_ref[0] + be_ref[0]
    return jnp.maximum(zn, 0.0)


def _layer_body(h_ref, agg_ref, deg_ref, ws_ref, wn_ref, bs_ref, bn_ref,
                g_ref, be_ref, o_ref):
    hn = _layer_math(h_ref, agg_ref, deg_ref, ws_ref, wn_ref, bs_ref, bn_ref,
                     g_ref, be_ref)
    for cc in range(NCHUNK):
        o_ref[cc] = hn[:, cc * CW:(cc + 1) * CW]


def _layer_last_body(h_ref, agg_ref, deg_ref, ws_ref, wn_ref, bs_ref, bn_ref,
                     g_ref, be_ref, wc_ref, bc_ref, o_ref):
    hn = _layer_math(h_ref, agg_ref, deg_ref, ws_ref, wn_ref, bs_ref, bn_ref,
                     g_ref, be_ref)
    o_ref[...] = jnp.dot(hn, wc_ref[...],
                         preferred_element_type=jnp.float32,
                         precision=lax.Precision.DEFAULT) + bc_ref[...]


def _common_specs(i):
    return [
        pl.BlockSpec((NCHUNK, MB, CW), lambda r: (0, r, 0)),
        pl.BlockSpec((NCHUNK, MB, CW), lambda r: (0, r, 0)),
        pl.BlockSpec((2, MB, CW), lambda r: (0, r, 0)),
        pl.BlockSpec((1, HID, HID), lambda r: (0, 0, 0)),
        pl.BlockSpec((1, HID, HID), lambda r: (0, 0, 0)),
        pl.BlockSpec((1, 1, HID), lambda r: (0, 0, 0)),
        pl.BlockSpec((1, 1, HID), lambda r: (0, 0, 0)),
        pl.BlockSpec((1, 1, HID), lambda r: (0, 0, 0)),
        pl.BlockSpec((1, 1, HID), lambda r: (0, 0, 0)),
    ]


def _layer(i, h, agg, degs, ws, wn, bs, bn, g, be):
    return pl.pallas_call(
        _layer_body,
        grid=(N // MB,),
        in_specs=_common_specs(i),
        out_specs=pl.BlockSpec((NCHUNK, MB, CW), lambda r: (0, r, 0)),
        out_shape=jax.ShapeDtypeStruct((NCHUNK, N, CW), jnp.float32),
    )(h, agg, degs, ws, wn, bs, bn, g, be)


def _layer_last(i, h, agg, degs, ws, wn, bs, bn, g, be, wc, bc):
    return pl.pallas_call(
        _layer_last_body,
        grid=(N // MB,),
        in_specs=_common_specs(i) + [
            pl.BlockSpec((HID, CW), lambda r: (0, 0)),
            pl.BlockSpec((1, CW), lambda r: (0, 0)),
        ],
        out_specs=pl.BlockSpec((MB, CW), lambda r: (r, 0)),
        out_shape=jax.ShapeDtypeStruct((N, CW), jnp.float32),
    )(h, agg, degs, ws, wn, bs, bn, g, be, wc, bc)


def kernel(x, edge_index, W_in, b_in, Ws_self, bs_self, Ws_neigh, bs_neigh,
           gammas, betas, W_cls, b_cls):
    f32 = jnp.float32
    src = edge_index[0].astype(jnp.int32).reshape(NTILES, TPT)
    dst = edge_index[1].astype(jnp.int32).reshape(NTILES, TPT)
    src3 = src[:, :NJ * EB].reshape(NTILES, NJ, EB)
    dst3 = dst[:, :NJ * EB].reshape(NTILES, NJ, EB)
    srct = src[:, NJ * EB:].reshape(NTILES, 1, TAIL)
    dstt = dst[:, NJ * EB:].reshape(NTILES, 1, TAIL)
    z128 = jnp.zeros((ZROWS, CW), f32)
    ones128 = jnp.ones((EB, CW), f32)

    dst32 = dst.reshape(32, E // 32)
    dstd = dst32[:, :NJD * EB].reshape(32, NJD, EB)
    dstdt = dst32[:, NJD * EB:].reshape(32, 1, TAILD)

    agg1 = _make_agg(True)
    aggn = _make_agg(False)

    hT = _in_proj(x, W_in, b_in.reshape(1, HID))
    degs = None
    out = None
    for i in range(LAYERS):
        if i == 0:
            agg, degs = agg1(hT, src3, dst3, srct, dstt, z128,
                             dstd, dstdt, ones128)
        else:
            agg = aggn(hT, src3, dst3, srct, dstt, z128)
        args = (i, hT, agg, degs, Ws_self[i][None], Ws_neigh[i][None],
                bs_self[i].reshape(1, 1, HID),
                bs_neigh[i].reshape(1, 1, HID),
                gammas[i].reshape(1, 1, HID),
                betas[i].reshape(1, 1, HID))
        if i < LAYERS - 1:
            hT = _layer(*args)
        else:
            wc = jnp.pad(W_cls, ((0, 0), (0, CW - NUM_CLASSES)))
            bc = jnp.pad(b_cls, (0, CW - NUM_CLASSES)).reshape(1, CW)
            out = _layer_last(*args, wc, bc)
    return out[:, :NUM_CLASSES]


# speedup vs baseline: 1.0004x; 1.0004x over previous
import functools

import jax
import jax.numpy as jnp
from jax import lax
from jax.experimental import pallas as pl
from jax.experimental.pallas import tpu as pltpu
from jax.experimental.pallas import tpu_sc as plsc

N = 10000
E = 160000
IN_DIM = 256
HID = 512
LAYERS = 3
NUM_CLASSES = 3

CW = 128
NCHUNK = HID // CW
EB = 128
NTILES = 16
TPT = E // NTILES
NJ = 78
TAIL = TPT - NJ * EB
NJD = 39
TAILD = E // 32 - NJD * EB
STRIPE = 632
LAST_STRIPE = N - 15 * STRIPE
ZROWS = 320
MB = 2000


def _agg_body(*refs, with_deg, with_agg=True):
    if with_deg and with_agg:
        (hT, src3, dst3, srct, dstt, z128, dstd, dstdt, ones128, agg, degs,
         src_v, dst_v, srct_v, dstt_v, gbuf, slab, sem,
         dstd_v, dstdt_v) = refs
    elif with_agg:
        (hT, src3, dst3, srct, dstt, z128, agg,
         src_v, dst_v, srct_v, dstt_v, gbuf, slab, sem) = refs
    else:
        (dstd, dstdt, z128, ones128, degs,
         gbuf, slab, sem, dstd_v, dstdt_v) = refs
    c = lax.axis_index("c")
    s = lax.axis_index("s")
    if with_agg:
        pltpu.sync_copy(src3.at[s], src_v)
        pltpu.sync_copy(dst3.at[s], dst_v)
        pltpu.sync_copy(srct.at[s], srct_v)
        pltpu.sync_copy(dstt.at[s], dstt_v)

    base = s * STRIPE

    def zero_stripes(slab):
        def zero(sz):
            def _z():
                pltpu.sync_copy(z128, slab.at[pl.ds(base, ZROWS)])
                pltpu.sync_copy(z128.at[pl.ds(0, sz - ZROWS)],
                                slab.at[pl.ds(base + ZROWS, sz - ZROWS)])
            return _z
        pl.when(s < 15)(zero(STRIPE))
        pl.when(s == 15)(zero(LAST_STRIPE))

    def copy_stripes(slab, out):
        def copyout(sz):
            return lambda: pltpu.sync_copy(slab.at[pl.ds(base, sz)],
                                           out.at[pl.ds(base, sz)])
        pl.when(s < 15)(copyout(STRIPE))
        pl.when(s == 15)(copyout(LAST_STRIPE))

    for cc in range(NCHUNK if with_agg else 0):
        @pl.when(cc % 2 == c)
        def _process(cc=cc):
            zero_stripes(slab)
            plsc.subcore_barrier()

            def step(j, carry):
                pltpu.async_copy(hT.at[cc].at[src_v.at[j]], gbuf,
                                 sem).wait()
                pltpu.sync_copy(gbuf, slab.at[dst_v.at[j]], add=True)
                return carry

            lax.fori_loop(0, NJ, step, 0)
            pltpu.async_copy(hT.at[cc].at[srct_v.at[0]],
                             gbuf.at[pl.ds(0, TAIL)], sem).wait()
            pltpu.sync_copy(gbuf.at[pl.ds(0, TAIL)],
                            slab.at[dstt_v.at[0]], add=True)
            plsc.subcore_barrier()

            copy_stripes(slab, agg.at[cc])
            plsc.subcore_barrier()

    if with_deg:
        wid = c * NTILES + s
        pltpu.sync_copy(dstd.at[wid], dstd_v)
        pltpu.sync_copy(dstdt.at[wid], dstdt_v)
        pltpu.sync_copy(ones128, gbuf)
        zero_stripes(slab)
        plsc.subcore_barrier()

        def dstep(j, carry):
            pltpu.sync_copy(gbuf, slab.at[dstd_v.at[j]], add=True)
            return carry

        lax.fori_loop(0, NJD, dstep, 0)
        pltpu.sync_copy(gbuf.at[pl.ds(0, TAILD)],
                        slab.at[dstdt_v.at[0]], add=True)
        plsc.subcore_barrier()
        copy_stripes(slab, degs.at[c])


def _make_agg(with_deg, with_agg=True):
    mesh = plsc.VectorSubcoreMesh(core_axis_name="c", subcore_axis_name="s")
    out_type = []
    scratch = []
    if with_agg:
        out_type.append(jax.ShapeDtypeStruct((NCHUNK, N, CW), jnp.float32))
        scratch += [
            pltpu.VMEM((NJ, EB), jnp.int32),
            pltpu.VMEM((NJ, EB), jnp.int32),
            pltpu.VMEM((1, TAIL), jnp.int32),
            pltpu.VMEM((1, TAIL), jnp.int32),
        ]
    scratch += [
        pltpu.VMEM((EB, CW), jnp.float32),
        pltpu.VMEM_SHARED((N, CW), jnp.float32),
        pltpu.SemaphoreType.DMA,
    ]
    if with_deg:
        out_type.append(jax.ShapeDtypeStruct((2, N, CW), jnp.float32))
        scratch.append(pltpu.VMEM((NJD, EB), jnp.int32))
        scratch.append(pltpu.VMEM((1, TAILD), jnp.int32))
    return pl.kernel(
        functools.partial(_agg_body, with_deg=with_deg, with_agg=with_agg),
        mesh=mesh,
        out_type=out_type if len(out_type) > 1 else out_type[0],
        scratch_types=scratch,
    )


def _in_proj_body(x_ref, w_ref, b_ref, o_ref):
    h = jnp.dot(x_ref[...], w_ref[...],
                preferred_element_type=jnp.float32,
                precision=lax.Precision.DEFAULT) + b_ref[...]
    for cc in range(NCHUNK):
        o_ref[cc] = h[:, cc * CW:(cc + 1) * CW]


def _in_proj(x, w, b2):
    return pl.pallas_call(
        _in_proj_body,
        grid=(N // MB,),
        in_specs=[
            pl.BlockSpec((MB, IN_DIM), lambda r: (r, 0)),
            pl.BlockSpec((IN_DIM, HID), lambda r: (0, 0)),
            pl.BlockSpec((1, HID), lambda r: (0, 0)),
        ],
        out_specs=pl.BlockSpec((NCHUNK, MB, CW), lambda r: (0, r, 0)),
        out_shape=jax.ShapeDtypeStruct((NCHUNK, N, CW), jnp.float32),
    )(x, w, b2)


def _layer_math(h_ref, agg_ref, deg_ref, ws_ref, wn_ref, bs_ref, bn_ref,
                g_ref, be_ref):
    deg = jnp.maximum(deg_ref[0][:, 0:1] + deg_ref[1][:, 0:1], 1.0)
    recip = 1.0 / deg
    wsv = ws_ref[0]
    wnv = wn_ref[0]
    z = bs_ref[0] + bn_ref[0]
    for cc in range(NCHUNK):
        z = z + jnp.dot(h_ref[cc], wsv[cc * CW:(cc + 1) * CW, :],
                        preferred_element_type=jnp.float32,
                        precision=lax.Precision.DEFAULT)
        z = z + jnp.dot(agg_ref[cc] * recip, wnv[cc * CW:(cc + 1) * CW, :],
                        preferred_element_type=jnp.float32,
                        precision=lax.Precision.DEFAULT)
    mu = jnp.mean(z, axis=1, keepdims=True)
    d = z - mu
    var = jnp.mean(d * d, axis=1, keepdims=True)
    zn = d * lax.rsqrt(var + 1e-5) * g_ref[0] + be_ref[0]
    return jnp.maximum(zn, 0.0)


def _layer_body(h_ref, agg_ref, deg_ref, ws_ref, wn_ref, bs_ref, bn_ref,
                g_ref, be_ref, o_ref):
    hn = _layer_math(h_ref, agg_ref, deg_ref, ws_ref, wn_ref, bs_ref, bn_ref,
                     g_ref, be_ref)
    for cc in range(NCHUNK):
        o_ref[cc] = hn[:, cc * CW:(cc + 1) * CW]


def _layer_last_body(h_ref, agg_ref, deg_ref, ws_ref, wn_ref, bs_ref, bn_ref,
                     g_ref, be_ref, wc_ref, bc_ref, o_ref):
    hn = _layer_math(h_ref, agg_ref, deg_ref, ws_ref, wn_ref, bs_ref, bn_ref,
                     g_ref, be_ref)
    o_ref[...] = jnp.dot(hn, wc_ref[...],
                         preferred_element_type=jnp.float32,
                         precision=lax.Precision.DEFAULT) + bc_ref[...]


def _common_specs(i):
    return [
        pl.BlockSpec((NCHUNK, MB, CW), lambda r: (0, r, 0)),
        pl.BlockSpec((NCHUNK, MB, CW), lambda r: (0, r, 0)),
        pl.BlockSpec((2, MB, CW), lambda r: (0, r, 0)),
        pl.BlockSpec((1, HID, HID), lambda r: (0, 0, 0)),
        pl.BlockSpec((1, HID, HID), lambda r: (0, 0, 0)),
        pl.BlockSpec((1, 1, HID), lambda r: (0, 0, 0)),
        pl.BlockSpec((1, 1, HID), lambda r: (0, 0, 0)),
        pl.BlockSpec((1, 1, HID), lambda r: (0, 0, 0)),
        pl.BlockSpec((1, 1, HID), lambda r: (0, 0, 0)),
    ]


def _layer(i, h, agg, degs, ws, wn, bs, bn, g, be):
    return pl.pallas_call(
        _layer_body,
        grid=(N // MB,),
        in_specs=_common_specs(i),
        out_specs=pl.BlockSpec((NCHUNK, MB, CW), lambda r: (0, r, 0)),
        out_shape=jax.ShapeDtypeStruct((NCHUNK, N, CW), jnp.float32),
    )(h, agg, degs, ws, wn, bs, bn, g, be)


def _layer_last(i, h, agg, degs, ws, wn, bs, bn, g, be, wc, bc):
    return pl.pallas_call(
        _layer_last_body,
        grid=(N // MB,),
        in_specs=_common_specs(i) + [
            pl.BlockSpec((HID, CW), lambda r: (0, 0)),
            pl.BlockSpec((1, CW), lambda r: (0, 0)),
        ],
        out_specs=pl.BlockSpec((MB, CW), lambda r: (r, 0)),
        out_shape=jax.ShapeDtypeStruct((N, CW), jnp.float32),
    )(h, agg, degs, ws, wn, bs, bn, g, be, wc, bc)


def kernel(x, edge_index, W_in, b_in, Ws_self, bs_self, Ws_neigh, bs_neigh,
           gammas, betas, W_cls, b_cls):
    f32 = jnp.float32
    src = edge_index[0].astype(jnp.int32).reshape(NTILES, TPT)
    dst = edge_index[1].astype(jnp.int32).reshape(NTILES, TPT)
    src3 = src[:, :NJ * EB].reshape(NTILES, NJ, EB)
    dst3 = dst[:, :NJ * EB].reshape(NTILES, NJ, EB)
    srct = src[:, NJ * EB:].reshape(NTILES, 1, TAIL)
    dstt = dst[:, NJ * EB:].reshape(NTILES, 1, TAIL)
    z128 = jnp.zeros((ZROWS, CW), f32)
    ones128 = jnp.ones((EB, CW), f32)

    dst32 = dst.reshape(32, E // 32)
    dstd = dst32[:, :NJD * EB].reshape(32, NJD, EB)
    dstdt = dst32[:, NJD * EB:].reshape(32, 1, TAILD)

    aggn = _make_agg(False)
    degs = _make_agg(True, with_agg=False)(dstd, dstdt, z128, ones128)

    hT = _in_proj(x, W_in, b_in.reshape(1, HID))
    out = None
    for i in range(LAYERS):
        agg = aggn(hT, src3, dst3, srct, dstt, z128)
        args = (i, hT, agg, degs, Ws_self[i][None], Ws_neigh[i][None],
                bs_self[i].reshape(1, 1, HID),
                bs_neigh[i].reshape(1, 1, HID),
                gammas[i].reshape(1, 1, HID),
                betas[i].reshape(1, 1, HID))
        if i < LAYERS - 1:
            hT = _layer(*args)
        else:
            wc = jnp.pad(W_cls, ((0, 0), (0, CW - NUM_CLASSES)))
            bc = jnp.pad(b_cls, (0, CW - NUM_CLASSES)).reshape(1, CW)
            out = _layer_last(*args, wc, bc)
    return out[:, :NUM_CLASSES]
